# baseline (speedup 1.0000x reference)
import jax
import jax.numpy as jnp
from jax import lax
from jax.experimental import pallas as pl
from jax.experimental.pallas import tpu as pltpu
from jax.experimental.pallas import tpu_sc as plsc

N_NODES = 10000
DIM = 128
N_EDGES = 320000
K = 50
CHUNK_ROWS = N_EDGES // K
N_SUBCORES = 16
ROWS_PER_TILE = CHUNK_ROWS // N_SUBCORES
SEG_PER_TILE = N_NODES // N_SUBCORES
NB = 20
N_BLOCKS = ROWS_PER_TILE // NB
DEPTH = 5
GAHEAD = 3
SLAG = 2


def _seg_kernel(item_emb, user_emb, zeros_hbm,
                ui0, ii0, ui1, ii1, ui2, ii2,
                out, acc, gidx_v, sidx_v, rows_v,
                gsem0, gsem1, gsem2, gsem3, gsem4,
                ssem0, ssem1, ssem2, ssem3, ssem4, isem):
    c = lax.axis_index("c")
    s = lax.axis_index("s")
    gsems = (gsem0, gsem1, gsem2, gsem3, gsem4)
    ssems = (ssem0, ssem1, ssem2, ssem3, ssem4)

    uidx = (ui0, ui1, ui2)
    iidx = (ii0, ii1, ii2)
    tasks = [(item_emb, iidx[b], uidx[b]) for b in range(3)]
    tasks += [(user_emb, uidx[b], iidx[b]) for b in range(3)]

    for t, (table, gidx, sidx) in enumerate(tasks):
        @pl.when(c == (t % 2))
        def _task(table=table, gidx=gidx, sidx=sidx, t=t):
            pltpu.sync_copy(zeros_hbm, acc.at[pl.ds(s * SEG_PER_TILE, SEG_PER_TILE)])
            plsc.subcore_barrier()

            tile_base = s * ROWS_PER_TILE
            pltpu.sync_copy(gidx.at[pl.ds(tile_base, NB)], gidx_v.at[0])
            pltpu.sync_copy(sidx.at[pl.ds(tile_base, NB)], sidx_v.at[0])
            for q in range(GAHEAD):
                pltpu.async_copy(table.at[gidx_v.at[0].at[q]],
                                 rows_v.at[q], gsems[q])

            def _block(bi, _):
                bp = lax.rem(bi, 2)

                @pl.when(bi < N_BLOCKS - 1)
                def _prefetch():
                    nxt = tile_base + (bi + 1) * NB
                    pltpu.async_copy(gidx.at[pl.ds(nxt, NB)], gidx_v.at[1 - bp], isem)
                    pltpu.async_copy(sidx.at[pl.ds(nxt, NB)], sidx_v.at[1 - bp], isem)

                for j in range(NB):
                    p = j % DEPTH
                    pltpu.make_async_copy(table.at[gidx_v.at[bp].at[j]],
                                          rows_v.at[p], gsems[p]).wait()
                    sp = (j - SLAG) % DEPTH

                    def _wait_scat(sp=sp, j=j):
                        pltpu.make_async_copy(
                            rows_v.at[sp], acc.at[sidx_v.at[bp].at[j]],
                            ssems[sp]).wait()

                    if j < SLAG:
                        pl.when(bi > 0)(_wait_scat)
                    else:
                        _wait_scat()
                    nj = j + GAHEAD
                    np_ = nj % DEPTH
                    if nj < NB:
                        pltpu.async_copy(table.at[gidx_v.at[bp].at[nj]],
                                         rows_v.at[np_], gsems[np_])
                    else:
                        if nj == NB:
                            @pl.when(bi < N_BLOCKS - 1)
                            def _wait_idx():
                                pltpu.make_async_copy(
                                    gidx.at[pl.ds(tile_base, NB)],
                                    gidx_v.at[1 - bp], isem).wait()
                                pltpu.make_async_copy(
                                    sidx.at[pl.ds(tile_base, NB)],
                                    sidx_v.at[1 - bp], isem).wait()

                        @pl.when(bi < N_BLOCKS - 1)
                        def _issue_next():
                            pltpu.async_copy(
                                table.at[gidx_v.at[1 - bp].at[nj - NB]],
                                rows_v.at[np_], gsems[np_])
                    pltpu.async_copy(rows_v.at[p], acc.at[sidx_v.at[bp].at[j]],
                                     ssems[p], add=True)
                return 0

            lax.fori_loop(0, N_BLOCKS, _block, 0)
            for q in range(SLAG):
                dp = (ROWS_PER_TILE - SLAG + q) % DEPTH
                pltpu.make_async_copy(rows_v.at[dp],
                                      acc.at[sidx_v.at[0].at[0]],
                                      ssems[dp]).wait()
            plsc.subcore_barrier()

            base = s * SEG_PER_TILE
            pltpu.sync_copy(acc.at[pl.ds(base, SEG_PER_TILE)],
                            out.at[t].at[pl.ds(base, SEG_PER_TILE)])


@jax.jit
def _segment_sums(item_emb, user_emb, zeros, idx_arrays):
    mesh = plsc.VectorSubcoreMesh(core_axis_name="c", subcore_axis_name="s")
    return pl.kernel(
        _seg_kernel,
        out_type=jax.ShapeDtypeStruct((6, N_NODES, DIM), jnp.float32),
        mesh=mesh,
        scratch_types=[
            pltpu.VMEM_SHARED((N_NODES, DIM), jnp.float32),
            pltpu.VMEM((2, NB, K), jnp.int32),
            pltpu.VMEM((2, NB, K), jnp.int32),
            pltpu.VMEM((DEPTH, K, DIM), jnp.float32),
        ] + [pltpu.SemaphoreType.DMA] * 11,
        compiler_params=pltpu.CompilerParams(use_tc_tiling_on_sc=False),
    )(item_emb, user_emb, zeros, *idx_arrays)


def _mm_kernel(aggs_ref, uw_ref, iw_ref, mu_ref, mi_ref, su_ref, si_ref):
    x = aggs_ref[:]
    uw = uw_ref[:]
    iw = iw_ref[:]
    yu = [jnp.dot(x[b], uw, preferred_element_type=jnp.float32)
          for b in range(3)]
    yi = [jnp.dot(x[3 + b], iw, preferred_element_type=jnp.float32)
          for b in range(3)]
    su_ref[:] = jax.nn.sigmoid(jnp.stack(yu, axis=0))
    si_ref[:] = jax.nn.sigmoid(jnp.stack(yi, axis=0))
    mu_ref[:] = jax.nn.sigmoid((yu[0] + yu[1] + yu[2]) * (1.0 / 3.0))
    mi_ref[:] = jax.nn.sigmoid((yi[0] + yi[1] + yi[2]) * (1.0 / 3.0))


BLK = 2000


@jax.jit
def _dense(aggs, u_w, i_w):
    grid = (N_NODES // BLK,)
    return pl.pallas_call(
        _mm_kernel,
        grid=grid,
        in_specs=[
            pl.BlockSpec((6, BLK, DIM), lambda j: (0, j, 0)),
            pl.BlockSpec((DIM, DIM), lambda j: (0, 0)),
            pl.BlockSpec((DIM, DIM), lambda j: (0, 0)),
        ],
        out_specs=[
            pl.BlockSpec((BLK, DIM), lambda j: (j, 0)),
            pl.BlockSpec((BLK, DIM), lambda j: (j, 0)),
            pl.BlockSpec((3, BLK, DIM), lambda j: (0, j, 0)),
            pl.BlockSpec((3, BLK, DIM), lambda j: (0, j, 0)),
        ],
        out_shape=[
            jax.ShapeDtypeStruct((N_NODES, DIM), jnp.float32),
            jax.ShapeDtypeStruct((N_NODES, DIM), jnp.float32),
            jax.ShapeDtypeStruct((3, N_NODES, DIM), jnp.float32),
            jax.ShapeDtypeStruct((3, N_NODES, DIM), jnp.float32),
        ],
    )(aggs, u_w, i_w)


def kernel(user_embedding, item_embedding, edge_index_b0, edge_index_b1,
           edge_index_b2, u_w, i_w):
    edges = (edge_index_b0, edge_index_b1, edge_index_b2)
    ui = [e[0].reshape(CHUNK_ROWS, K) for e in edges]
    ii = [e[1].reshape(CHUNK_ROWS, K) for e in edges]
    ordered = (ui[0], ii[0], ui[1], ii[1], ui[2], ii[2])

    zeros = jnp.zeros((SEG_PER_TILE, DIM), jnp.float32)
    aggs = _segment_sums(item_embedding, user_embedding, zeros, ordered)
    mu, mi, su, si = _dense(aggs, u_w, i_w)
    return (mu, mi, su, si)


# speedup vs baseline: 1.0939x; 1.0939x over previous
import jax
import jax.numpy as jnp
from jax import lax
from jax.experimental import pallas as pl
from jax.experimental.pallas import tpu as pltpu
from jax.experimental.pallas import tpu_sc as plsc

N_NODES = 10000
DIM = 128
N_EDGES = 320000
K = 50
CHUNK_ROWS = N_EDGES // K
N_SUBCORES = 16
ROWS_PER_TILE = CHUNK_ROWS // N_SUBCORES
SEG_PER_TILE = N_NODES // N_SUBCORES
NB = 20
N_BLOCKS = ROWS_PER_TILE // NB
DEPTH = 5
GAHEAD = 4


def _seg_kernel(item_emb, user_emb, zeros_hbm,
                ui0, ii0, ui1, ii1, ui2, ii2,
                out, acc, gidx_v, sidx_v, rows_v,
                gsem0, gsem1, gsem2, gsem3, gsem4, isem):
    c = lax.axis_index("c")
    s = lax.axis_index("s")
    gsems = (gsem0, gsem1, gsem2, gsem3, gsem4)

    uidx = (ui0, ui1, ui2)
    iidx = (ii0, ii1, ii2)
    tasks = [(item_emb, iidx[b], uidx[b]) for b in range(3)]
    tasks += [(user_emb, uidx[b], iidx[b]) for b in range(3)]

    for t, (table, gidx, sidx) in enumerate(tasks):
        @pl.when(c == (t % 2))
        def _task(table=table, gidx=gidx, sidx=sidx, t=t):
            pltpu.sync_copy(zeros_hbm, acc.at[pl.ds(s * SEG_PER_TILE, SEG_PER_TILE)])
            plsc.subcore_barrier()

            tile_base = s * ROWS_PER_TILE
            pltpu.sync_copy(gidx.at[pl.ds(tile_base, NB)], gidx_v.at[0])
            pltpu.sync_copy(sidx.at[pl.ds(tile_base, NB)], sidx_v.at[0])
            for q in range(GAHEAD):
                pltpu.async_copy(table.at[gidx_v.at[0].at[q]],
                                 rows_v.at[q], gsems[q])

            def _block(bi, _):
                bp = lax.rem(bi, 2)

                @pl.when(bi < N_BLOCKS - 1)
                def _prefetch():
                    nxt = tile_base + (bi + 1) * NB
                    pltpu.async_copy(gidx.at[pl.ds(nxt, NB)], gidx_v.at[1 - bp], isem)
                    pltpu.async_copy(sidx.at[pl.ds(nxt, NB)], sidx_v.at[1 - bp], isem)

                for j in range(NB):
                    p = j % DEPTH
                    pltpu.make_async_copy(table.at[gidx_v.at[bp].at[j]],
                                          rows_v.at[p], gsems[p]).wait()
                    nj = j + GAHEAD
                    np_ = nj % DEPTH
                    if nj < NB:
                        pltpu.async_copy(table.at[gidx_v.at[bp].at[nj]],
                                         rows_v.at[np_], gsems[np_])
                    else:
                        if nj == NB:
                            @pl.when(bi < N_BLOCKS - 1)
                            def _wait_idx():
                                pltpu.make_async_copy(
                                    gidx.at[pl.ds(tile_base, NB)],
                                    gidx_v.at[1 - bp], isem).wait()
                                pltpu.make_async_copy(
                                    sidx.at[pl.ds(tile_base, NB)],
                                    sidx_v.at[1 - bp], isem).wait()

                        @pl.when(bi < N_BLOCKS - 1)
                        def _issue_next():
                            pltpu.async_copy(
                                table.at[gidx_v.at[1 - bp].at[nj - NB]],
                                rows_v.at[np_], gsems[np_])
                    pltpu.sync_copy(rows_v.at[p], acc.at[sidx_v.at[bp].at[j]],
                                    add=True)
                return 0

            lax.fori_loop(0, N_BLOCKS, _block, 0)
            plsc.subcore_barrier()

            base = s * SEG_PER_TILE
            pltpu.sync_copy(acc.at[pl.ds(base, SEG_PER_TILE)],
                            out.at[t].at[pl.ds(base, SEG_PER_TILE)])


@jax.jit
def _segment_sums(item_emb, user_emb, zeros, idx_arrays):
    mesh = plsc.VectorSubcoreMesh(core_axis_name="c", subcore_axis_name="s")
    return pl.kernel(
        _seg_kernel,
        out_type=jax.ShapeDtypeStruct((6, N_NODES, DIM), jnp.float32),
        mesh=mesh,
        scratch_types=[
            pltpu.VMEM_SHARED((N_NODES, DIM), jnp.float32),
            pltpu.VMEM((2, NB, K), jnp.int32),
            pltpu.VMEM((2, NB, K), jnp.int32),
            pltpu.VMEM((DEPTH, K, DIM), jnp.float32),
        ] + [pltpu.SemaphoreType.DMA] * 6,
        compiler_params=pltpu.CompilerParams(use_tc_tiling_on_sc=False),
    )(item_emb, user_emb, zeros, *idx_arrays)


def _mm_kernel(aggs_ref, uw_ref, iw_ref, mu_ref, mi_ref, su_ref, si_ref):
    x = aggs_ref[:]
    uw = uw_ref[:]
    iw = iw_ref[:]
    yu = [jnp.dot(x[b], uw, preferred_element_type=jnp.float32)
          for b in range(3)]
    yi = [jnp.dot(x[3 + b], iw, preferred_element_type=jnp.float32)
          for b in range(3)]
    su_ref[:] = jax.nn.sigmoid(jnp.stack(yu, axis=0))
    si_ref[:] = jax.nn.sigmoid(jnp.stack(yi, axis=0))
    mu_ref[:] = jax.nn.sigmoid((yu[0] + yu[1] + yu[2]) * (1.0 / 3.0))
    mi_ref[:] = jax.nn.sigmoid((yi[0] + yi[1] + yi[2]) * (1.0 / 3.0))


BLK = 2000


@jax.jit
def _dense(aggs, u_w, i_w):
    grid = (N_NODES // BLK,)
    return pl.pallas_call(
        _mm_kernel,
        grid=grid,
        in_specs=[
            pl.BlockSpec((6, BLK, DIM), lambda j: (0, j, 0)),
            pl.BlockSpec((DIM, DIM), lambda j: (0, 0)),
            pl.BlockSpec((DIM, DIM), lambda j: (0, 0)),
        ],
        out_specs=[
            pl.BlockSpec((BLK, DIM), lambda j: (j, 0)),
            pl.BlockSpec((BLK, DIM), lambda j: (j, 0)),
            pl.BlockSpec((3, BLK, DIM), lambda j: (0, j, 0)),
            pl.BlockSpec((3, BLK, DIM), lambda j: (0, j, 0)),
        ],
        out_shape=[
            jax.ShapeDtypeStruct((N_NODES, DIM), jnp.float32),
            jax.ShapeDtypeStruct((N_NODES, DIM), jnp.float32),
            jax.ShapeDtypeStruct((3, N_NODES, DIM), jnp.float32),
            jax.ShapeDtypeStruct((3, N_NODES, DIM), jnp.float32),
        ],
    )(aggs, u_w, i_w)


def kernel(user_embedding, item_embedding, edge_index_b0, edge_index_b1,
           edge_index_b2, u_w, i_w):
    edges = (edge_index_b0, edge_index_b1, edge_index_b2)
    ui = [e[0].reshape(CHUNK_ROWS, K) for e in edges]
    ii = [e[1].reshape(CHUNK_ROWS, K) for e in edges]
    ordered = (ui[0], ii[0], ui[1], ii[1], ui[2], ii[2])

    zeros = jnp.zeros((SEG_PER_TILE, DIM), jnp.float32)
    aggs = _segment_sums(item_embedding, user_embedding, zeros, ordered)
    mu, mi, su, si = _dense(aggs, u_w, i_w)
    return (mu, mi, su, si)


# speedup vs baseline: 1.1017x; 1.0071x over previous
import jax
import jax.numpy as jnp
from jax import lax
from jax.experimental import pallas as pl
from jax.experimental.pallas import tpu as pltpu
from jax.experimental.pallas import tpu_sc as plsc

N_NODES = 10000
DIM = 128
N_EDGES = 320000
K = 50
CHUNK_ROWS = N_EDGES // K
N_SUBCORES = 16
ROWS_PER_TILE = CHUNK_ROWS // N_SUBCORES
SEG_PER_TILE = N_NODES // N_SUBCORES
NB = 20
N_BLOCKS = ROWS_PER_TILE // NB
DEPTH = 5
GAHEAD = 4


def _seg_kernel(item_emb, user_emb, zeros_hbm,
                ui0, ii0, ui1, ii1, ui2, ii2,
                out, acc, gidx_v, sidx_v, rows_v,
                gsem0, gsem1, gsem2, gsem3, gsem4, isem):
    c = lax.axis_index("c")
    s = lax.axis_index("s")
    gsems = (gsem0, gsem1, gsem2, gsem3, gsem4)

    uidx = (ui0, ui1, ui2)
    iidx = (ii0, ii1, ii2)
    tasks = [(item_emb, iidx[b], uidx[b]) for b in range(3)]
    tasks += [(user_emb, uidx[b], iidx[b]) for b in range(3)]

    tile_base = s * ROWS_PER_TILE
    seg_base = s * SEG_PER_TILE

    def _prime(table, gidx, sidx):
        pltpu.sync_copy(gidx.at[pl.ds(tile_base, NB)], gidx_v.at[0])
        pltpu.sync_copy(sidx.at[pl.ds(tile_base, NB)], sidx_v.at[0])
        for q in range(GAHEAD):
            pltpu.async_copy(table.at[gidx_v.at[0].at[q]],
                             rows_v.at[q], gsems[q])

    def _chunk_loop(table, gidx, sidx):
        def _block(bi, _):
                bp = lax.rem(bi, 2)

                @pl.when(bi < N_BLOCKS - 1)
                def _prefetch():
                    nxt = tile_base + (bi + 1) * NB
                    pltpu.async_copy(gidx.at[pl.ds(nxt, NB)], gidx_v.at[1 - bp], isem)
                    pltpu.async_copy(sidx.at[pl.ds(nxt, NB)], sidx_v.at[1 - bp], isem)

                for j in range(NB):
                    p = j % DEPTH
                    pltpu.make_async_copy(table.at[gidx_v.at[bp].at[j]],
                                          rows_v.at[p], gsems[p]).wait()
                    nj = j + GAHEAD
                    np_ = nj % DEPTH
                    if nj < NB:
                        pltpu.async_copy(table.at[gidx_v.at[bp].at[nj]],
                                         rows_v.at[np_], gsems[np_])
                    else:
                        if nj == NB:
                            @pl.when(bi < N_BLOCKS - 1)
                            def _wait_idx():
                                pltpu.make_async_copy(
                                    gidx.at[pl.ds(tile_base, NB)],
                                    gidx_v.at[1 - bp], isem).wait()
                                pltpu.make_async_copy(
                                    sidx.at[pl.ds(tile_base, NB)],
                                    sidx_v.at[1 - bp], isem).wait()

                        @pl.when(bi < N_BLOCKS - 1)
                        def _issue_next():
                            pltpu.async_copy(
                                table.at[gidx_v.at[1 - bp].at[nj - NB]],
                                rows_v.at[np_], gsems[np_])
                    pltpu.sync_copy(rows_v.at[p], acc.at[sidx_v.at[bp].at[j]],
                                    add=True)
                return 0

        lax.fori_loop(0, N_BLOCKS, _block, 0)

    for par in range(2):
        mine = [(t, tasks[t]) for t in range(6) if t % 2 == par]

        @pl.when(c == par)
        def _core(mine=mine):
            _prime(*mine[0][1])
            pltpu.sync_copy(zeros_hbm, acc.at[pl.ds(seg_base, SEG_PER_TILE)])
            plsc.subcore_barrier()
            for k, (t, (table, gidx, sidx)) in enumerate(mine):
                _chunk_loop(table, gidx, sidx)
                plsc.subcore_barrier()
                if k < len(mine) - 1:
                    _prime(*mine[k + 1][1])
                    pltpu.sync_copy(acc.at[pl.ds(seg_base, SEG_PER_TILE)],
                                    out.at[t].at[pl.ds(seg_base, SEG_PER_TILE)])
                    pltpu.sync_copy(zeros_hbm,
                                    acc.at[pl.ds(seg_base, SEG_PER_TILE)])
                    plsc.subcore_barrier()
                else:
                    pltpu.sync_copy(acc.at[pl.ds(seg_base, SEG_PER_TILE)],
                                    out.at[t].at[pl.ds(seg_base, SEG_PER_TILE)])


@jax.jit
def _segment_sums(item_emb, user_emb, zeros, idx_arrays):
    mesh = plsc.VectorSubcoreMesh(core_axis_name="c", subcore_axis_name="s")
    return pl.kernel(
        _seg_kernel,
        out_type=jax.ShapeDtypeStruct((6, N_NODES, DIM), jnp.float32),
        mesh=mesh,
        scratch_types=[
            pltpu.VMEM_SHARED((N_NODES, DIM), jnp.float32),
            pltpu.VMEM((2, NB, K), jnp.int32),
            pltpu.VMEM((2, NB, K), jnp.int32),
            pltpu.VMEM((DEPTH, K, DIM), jnp.float32),
        ] + [pltpu.SemaphoreType.DMA] * 6,
        compiler_params=pltpu.CompilerParams(use_tc_tiling_on_sc=False),
    )(item_emb, user_emb, zeros, *idx_arrays)


def _mm_kernel(aggs_ref, uw_ref, iw_ref, mu_ref, mi_ref, su_ref, si_ref):
    x = aggs_ref[:]
    uw = uw_ref[:]
    iw = iw_ref[:]
    yu = [jnp.dot(x[b], uw, preferred_element_type=jnp.float32)
          for b in range(3)]
    yi = [jnp.dot(x[3 + b], iw, preferred_element_type=jnp.float32)
          for b in range(3)]
    su_ref[:] = jax.nn.sigmoid(jnp.stack(yu, axis=0))
    si_ref[:] = jax.nn.sigmoid(jnp.stack(yi, axis=0))
    mu_ref[:] = jax.nn.sigmoid((yu[0] + yu[1] + yu[2]) * (1.0 / 3.0))
    mi_ref[:] = jax.nn.sigmoid((yi[0] + yi[1] + yi[2]) * (1.0 / 3.0))


BLK = 2000


@jax.jit
def _dense(aggs, u_w, i_w):
    grid = (N_NODES // BLK,)
    return pl.pallas_call(
        _mm_kernel,
        grid=grid,
        in_specs=[
            pl.BlockSpec((6, BLK, DIM), lambda j: (0, j, 0)),
            pl.BlockSpec((DIM, DIM), lambda j: (0, 0)),
            pl.BlockSpec((DIM, DIM), lambda j: (0, 0)),
        ],
        out_specs=[
            pl.BlockSpec((BLK, DIM), lambda j: (j, 0)),
            pl.BlockSpec((BLK, DIM), lambda j: (j, 0)),
            pl.BlockSpec((3, BLK, DIM), lambda j: (0, j, 0)),
            pl.BlockSpec((3, BLK, DIM), lambda j: (0, j, 0)),
        ],
        out_shape=[
            jax.ShapeDtypeStruct((N_NODES, DIM), jnp.float32),
            jax.ShapeDtypeStruct((N_NODES, DIM), jnp.float32),
            jax.ShapeDtypeStruct((3, N_NODES, DIM), jnp.float32),
            jax.ShapeDtypeStruct((3, N_NODES, DIM), jnp.float32),
        ],
    )(aggs, u_w, i_w)


def kernel(user_embedding, item_embedding, edge_index_b0, edge_index_b1,
           edge_index_b2, u_w, i_w):
    edges = (edge_index_b0, edge_index_b1, edge_index_b2)
    ui = [e[0].reshape(CHUNK_ROWS, K) for e in edges]
    ii = [e[1].reshape(CHUNK_ROWS, K) for e in edges]
    ordered = (ui[0], ii[0], ui[1], ii[1], ui[2], ii[2])

    zeros = jnp.zeros((SEG_PER_TILE, DIM), jnp.float32)
    aggs = _segment_sums(item_embedding, user_embedding, zeros, ordered)
    mu, mi, su, si = _dense(aggs, u_w, i_w)
    return (mu, mi, su, si)
